# initial kernel scaffold (unmeasured)
import jax
import jax.numpy as jnp
from jax import lax
from jax.experimental import pallas as pl
from jax.experimental.pallas import tpu as pltpu

N_DEV = 4
B = 2
S = 512
HQ = 8
DH = 64
HD = HQ * DH
BLK = 64
SCALE = 0.125


def kernel(x, Wq, K_ext, V_ext, Wo):
    K2 = K_ext.reshape(B, S, HD)
    V2 = V_ext.reshape(B, S, HD)

    def body(x_ref, wq_ref, k_ref, v_ref, wo_ref, out_ref,
             comm_k, comm_v, q_buf, ctx_buf,
             send_k, recv_k, send_v, recv_v):
        my = lax.axis_index("i")
        left = (my + N_DEV - 1) % N_DEV
        right = (my + 1) % N_DEV

        barrier = pltpu.get_barrier_semaphore()
        for nbr in (left, right):
            pl.semaphore_signal(barrier, inc=1, device_id=(nbr,),
                                device_id_type=pl.DeviceIdType.MESH)
        pl.semaphore_wait(barrier, 2)

        for b in range(B):
            q_buf[b] = jnp.dot(x_ref[b], wq_ref[...],
                               preferred_element_type=jnp.float32)

        comm_k[0] = k_ref[...]
        comm_v[0] = v_ref[...]

        for h in range(N_DEV - 1):
            rk = pltpu.make_async_remote_copy(
                src_ref=comm_k.at[h], dst_ref=comm_k.at[h + 1],
                send_sem=send_k.at[h], recv_sem=recv_k.at[h],
                device_id=(right,), device_id_type=pl.DeviceIdType.MESH)
            rv = pltpu.make_async_remote_copy(
                src_ref=comm_v.at[h], dst_ref=comm_v.at[h + 1],
                send_sem=send_v.at[h], recv_sem=recv_v.at[h],
                device_id=(right,), device_id_type=pl.DeviceIdType.MESH)
            rk.start()
            rv.start()
            rk.wait()
            rv.wait()

        row_blk = lax.broadcasted_iota(jnp.int32, (S, S), 0) // BLK
        col_blk = lax.broadcasted_iota(jnp.int32, (S, S), 1) // BLK
        diag_mask = (col_blk <= row_blk).astype(jnp.float32)

        for b in range(B):
            for hh in range(HQ):
                lo = hh * DH
                q = q_buf[b, :, lo:lo + DH]
                ctx = jnp.zeros((S, DH), jnp.float32)
                den = jnp.zeros((S, 1), jnp.float32)
                for s in range(N_DEV):
                    kk = comm_k[s, b, :, lo:lo + DH]
                    vv = comm_v[s, b, :, lo:lo + DH]
                    sc = lax.dot_general(
                        q, kk, (((1,), (1,)), ((), ())),
                        preferred_element_type=jnp.float32) * SCALE
                    w = jnp.exp(sc)
                    if s == 0:
                        w = w * diag_mask
                    else:
                        w = w * (s <= my).astype(jnp.float32)
                    ctx = ctx + jnp.dot(w, vv,
                                        preferred_element_type=jnp.float32)
                    den = den + jnp.sum(w, axis=1, keepdims=True)
                ctx_buf[b, :, lo:lo + DH] = ctx / den

        for b in range(B):
            out_ref[b] = jnp.dot(ctx_buf[b], wo_ref[...],
                                 preferred_element_type=jnp.float32)

    return pl.pallas_call(
        body,
        out_shape=jax.ShapeDtypeStruct((B, S, 768), jnp.float32),
        in_specs=[pl.BlockSpec(memory_space=pltpu.VMEM)] * 5,
        out_specs=pl.BlockSpec(memory_space=pltpu.VMEM),
        scratch_shapes=[
            pltpu.VMEM((N_DEV, B, S, HD), jnp.float32),
            pltpu.VMEM((N_DEV, B, S, HD), jnp.float32),
            pltpu.VMEM((B, S, HD), jnp.float32),
            pltpu.VMEM((B, S, HD), jnp.float32),
            pltpu.SemaphoreType.DMA((N_DEV - 1,)),
            pltpu.SemaphoreType.DMA((N_DEV - 1,)),
            pltpu.SemaphoreType.DMA((N_DEV - 1,)),
            pltpu.SemaphoreType.DMA((N_DEV - 1,)),
        ],
        compiler_params=pltpu.CompilerParams(collective_id=0),
    )(x, Wq, K2, V2, Wo)


# baseline (device time: 179939 ns/iter reference)
import jax
import jax.numpy as jnp
from jax import lax
from jax.experimental import pallas as pl
from jax.experimental.pallas import tpu as pltpu

N_DEV = 4
B = 2
S = 512
HQ = 8
DH = 64
HD = HQ * DH
BLK = 64
SCALE = 0.125


def kernel(x, Wq, K_ext, V_ext, Wo):
    K2 = K_ext.reshape(B, S, HD)
    V2 = V_ext.reshape(B, S, HD)

    def body(x_ref, wq_ref, k_ref, v_ref, wo_ref, out_ref,
             comm_k, comm_v, q_buf, ctx_buf,
             send_k, recv_k, send_v, recv_v):
        my = lax.axis_index("i")
        left = (my + N_DEV - 1) % N_DEV
        right = (my + 1) % N_DEV

        barrier = pltpu.get_barrier_semaphore()
        for nbr in (left, right):
            pl.semaphore_signal(barrier, inc=1, device_id=(nbr,),
                                device_id_type=pl.DeviceIdType.MESH)
        pl.semaphore_wait(barrier, 2)

        for b in range(B):
            q_buf[b] = jnp.dot(x_ref[b], wq_ref[...],
                               preferred_element_type=jnp.float32)

        comm_k[0] = k_ref[...]
        comm_v[0] = v_ref[...]

        for h in range(N_DEV - 1):
            rk = pltpu.make_async_remote_copy(
                src_ref=comm_k.at[h], dst_ref=comm_k.at[h + 1],
                send_sem=send_k.at[h], recv_sem=recv_k.at[h],
                device_id=(right,), device_id_type=pl.DeviceIdType.MESH)
            rv = pltpu.make_async_remote_copy(
                src_ref=comm_v.at[h], dst_ref=comm_v.at[h + 1],
                send_sem=send_v.at[h], recv_sem=recv_v.at[h],
                device_id=(right,), device_id_type=pl.DeviceIdType.MESH)
            rk.start()
            rv.start()
            rk.wait()
            rv.wait()

        row_blk = lax.broadcasted_iota(jnp.int32, (S, S), 0) // BLK
        col_blk = lax.broadcasted_iota(jnp.int32, (S, S), 1) // BLK
        diag_mask = (col_blk <= row_blk).astype(jnp.float32)

        for b in range(B):
            for hh in range(HQ):
                lo = hh * DH
                q = q_buf[b, :, lo:lo + DH]
                ctx = jnp.zeros((S, DH), jnp.float32)
                den = jnp.zeros((S, 1), jnp.float32)
                for s in range(N_DEV):
                    kk = comm_k[s, b, :, lo:lo + DH]
                    vv = comm_v[s, b, :, lo:lo + DH]
                    sc = lax.dot_general(
                        q, kk, (((1,), (1,)), ((), ())),
                        preferred_element_type=jnp.float32) * SCALE
                    w = jnp.exp(sc)
                    if s == 0:
                        w = w * diag_mask
                    else:
                        w = w * (s <= my).astype(jnp.float32)
                    ctx = ctx + jnp.dot(w, vv,
                                        preferred_element_type=jnp.float32)
                    den = den + jnp.sum(w, axis=1, keepdims=True)
                ctx_buf[b, :, lo:lo + DH] = ctx / den

        for b in range(B):
            out_ref[b] = jnp.dot(ctx_buf[b], wo_ref[...],
                                 preferred_element_type=jnp.float32)

    return pl.pallas_call(
        body,
        out_shape=jax.ShapeDtypeStruct((B, S, 768), jnp.float32),
        in_specs=[pl.BlockSpec(memory_space=pltpu.VMEM)] * 5,
        out_specs=pl.BlockSpec(memory_space=pltpu.VMEM),
        scratch_shapes=[
            pltpu.VMEM((N_DEV, B, S, HD), jnp.float32),
            pltpu.VMEM((N_DEV, B, S, HD), jnp.float32),
            pltpu.VMEM((B, S, HD), jnp.float32),
            pltpu.VMEM((B, S, HD), jnp.float32),
            pltpu.SemaphoreType.DMA((N_DEV - 1,)),
            pltpu.SemaphoreType.DMA((N_DEV - 1,)),
            pltpu.SemaphoreType.DMA((N_DEV - 1,)),
            pltpu.SemaphoreType.DMA((N_DEV - 1,)),
        ],
        compiler_params=pltpu.CompilerParams(
            collective_id=0,
            vmem_limit_bytes=100 * 1024 * 1024,
        ),
    )(x, Wq, K2, V2, Wo)


# device time: 103235 ns/iter; 1.7430x vs baseline; 1.7430x over previous
import jax
import jax.numpy as jnp
from jax import lax
from jax.experimental import pallas as pl
from jax.experimental.pallas import tpu as pltpu

N_DEV = 4
B = 2
S = 512
HQ = 8
DH = 64
HD = HQ * DH
BLK = 64
SCALE = 0.125


def kernel(x, Wq, K_ext, V_ext, Wo):
    K2 = K_ext.reshape(B, S, HD)
    V2 = V_ext.reshape(B, S, HD)

    def body(x_ref, wq_ref, k_ref, v_ref, wo_ref, out_ref,
             comm_k, comm_v, q_buf, ctx_buf, acc, den,
             send_k, recv_k, send_v, recv_v):
        my = lax.axis_index("i")
        left = (my + N_DEV - 1) % N_DEV
        right = (my + 1) % N_DEV

        barrier = pltpu.get_barrier_semaphore()
        for nbr in (left, right):
            pl.semaphore_signal(barrier, inc=1, device_id=(nbr,),
                                device_id_type=pl.DeviceIdType.MESH)
        pl.semaphore_wait(barrier, 2)

        comm_k[0] = k_ref[...]
        comm_v[0] = v_ref[...]

        def rdma(ref, sems_s, sems_r, i, src, dst, target):
            return pltpu.make_async_remote_copy(
                src_ref=src, dst_ref=dst,
                send_sem=sems_s.at[i], recv_sem=sems_r.at[i],
                device_id=(target,), device_id_type=pl.DeviceIdType.MESH)

        kA = rdma(comm_k, send_k, recv_k, 0, comm_k.at[0], comm_k.at[1], right)
        kB = rdma(comm_k, send_k, recv_k, 1, comm_k.at[0], comm_k.at[3], left)
        vA = rdma(comm_v, send_v, recv_v, 0, comm_v.at[0], comm_v.at[1], right)
        vB = rdma(comm_v, send_v, recv_v, 1, comm_v.at[0], comm_v.at[3], left)
        kA.start(); vA.start(); kB.start(); vB.start()

        row_blk = lax.broadcasted_iota(jnp.int32, (S, S), 0) // BLK
        col_blk = lax.broadcasted_iota(jnp.int32, (S, S), 1) // BLK
        diag_mask = (col_blk <= row_blk).astype(jnp.float32)

        for b in range(B):
            q_buf[b] = jnp.dot(x_ref[b], wq_ref[...],
                               preferred_element_type=jnp.float32)

        def attend(slot, keep=None, masked=False, init=False):
            for b in range(B):
                for hh in range(HQ):
                    lo = hh * DH
                    q = q_buf[b, :, lo:lo + DH]
                    kk = comm_k[slot, b, :, lo:lo + DH]
                    vv = comm_v[slot, b, :, lo:lo + DH]
                    sc = lax.dot_general(
                        q, kk, (((1,), (1,)), ((), ())),
                        preferred_element_type=jnp.float32) * SCALE
                    w = jnp.exp(sc)
                    if masked:
                        w = w * diag_mask
                    if keep is not None:
                        w = w * keep
                    pv = jnp.dot(w, vv, preferred_element_type=jnp.float32)
                    ds = jnp.sum(w, axis=1, keepdims=True)
                    if init:
                        acc[b, :, lo:lo + DH] = pv
                        den[b, hh] = ds
                    else:
                        acc[b, :, lo:lo + DH] = acc[b, :, lo:lo + DH] + pv
                        den[b, hh] = den[b, hh] + ds

        attend(0, masked=True, init=True)

        kA.wait_recv(); vA.wait_recv(); kB.wait_recv(); vB.wait_recv()
        kC = rdma(comm_k, send_k, recv_k, 2,
                  comm_k.at[1, 0], comm_k.at[2, 0], right)
        kD = rdma(comm_k, send_k, recv_k, 3,
                  comm_k.at[3, 1], comm_k.at[2, 1], left)
        vC = rdma(comm_v, send_v, recv_v, 2,
                  comm_v.at[1, 0], comm_v.at[2, 0], right)
        vD = rdma(comm_v, send_v, recv_v, 3,
                  comm_v.at[3, 1], comm_v.at[2, 1], left)
        kC.start(); vC.start(); kD.start(); vD.start()

        attend(1, keep=(my >= 1).astype(jnp.float32))
        attend(3, keep=(my == 3).astype(jnp.float32))

        kC.wait_recv(); vC.wait_recv(); kD.wait_recv(); vD.wait_recv()
        attend(2, keep=(my >= 2).astype(jnp.float32))

        for b in range(B):
            for hh in range(HQ):
                lo = hh * DH
                ctx_buf[b, :, lo:lo + DH] = acc[b, :, lo:lo + DH] / den[b, hh]
            out_ref[b] = jnp.dot(ctx_buf[b], wo_ref[...],
                                 preferred_element_type=jnp.float32)

        kA.wait_send(); vA.wait_send(); kB.wait_send(); vB.wait_send()
        kC.wait_send(); vC.wait_send(); kD.wait_send(); vD.wait_send()

    return pl.pallas_call(
        body,
        out_shape=jax.ShapeDtypeStruct((B, S, 768), jnp.float32),
        in_specs=[pl.BlockSpec(memory_space=pltpu.VMEM)] * 5,
        out_specs=pl.BlockSpec(memory_space=pltpu.VMEM),
        scratch_shapes=[
            pltpu.VMEM((N_DEV, B, S, HD), jnp.float32),
            pltpu.VMEM((N_DEV, B, S, HD), jnp.float32),
            pltpu.VMEM((B, S, HD), jnp.float32),
            pltpu.VMEM((B, S, HD), jnp.float32),
            pltpu.VMEM((B, S, HD), jnp.float32),
            pltpu.VMEM((B, HQ, S, 1), jnp.float32),
            pltpu.SemaphoreType.DMA((4,)),
            pltpu.SemaphoreType.DMA((4,)),
            pltpu.SemaphoreType.DMA((4,)),
            pltpu.SemaphoreType.DMA((4,)),
        ],
        compiler_params=pltpu.CompilerParams(
            collective_id=0,
            vmem_limit_bytes=100 * 1024 * 1024,
        ),
    )(x, Wq, K2, V2, Wo)


# device time: 69251 ns/iter; 2.5984x vs baseline; 1.4907x over previous
import jax
import jax.numpy as jnp
from jax import lax
from jax.experimental import pallas as pl
from jax.experimental.pallas import tpu as pltpu

N_DEV = 4
B = 2
S = 512
HQ = 8
DH = 64
HD = HQ * DH
BLK = 64
SCALE = 0.125


def kernel(x, Wq, K_ext, V_ext, Wo):
    K2 = K_ext.reshape(B, S, HD)
    V2 = V_ext.reshape(B, S, HD)

    def body(x_ref, wq_ref, k_ref, v_ref, wo_ref, out_ref,
             comm_k, comm_v, q_buf, ctx_buf, acc, den,
             send_k, recv_k, send_v, recv_v):
        my = lax.axis_index("i")
        left = (my + N_DEV - 1) % N_DEV
        right = (my + 1) % N_DEV

        barrier = pltpu.get_barrier_semaphore()
        for nbr in (left, right):
            pl.semaphore_signal(barrier, inc=1, device_id=(nbr,),
                                device_id_type=pl.DeviceIdType.MESH)
        pl.semaphore_wait(barrier, 2)

        comm_k[0] = k_ref[...].astype(jnp.bfloat16)
        comm_v[0] = v_ref[...].astype(jnp.bfloat16)

        def rdma(sems_s, sems_r, i, src, dst, target):
            return pltpu.make_async_remote_copy(
                src_ref=src, dst_ref=dst,
                send_sem=sems_s.at[i], recv_sem=sems_r.at[i],
                device_id=(target,), device_id_type=pl.DeviceIdType.MESH)

        kA = rdma(send_k, recv_k, 0, comm_k.at[0], comm_k.at[1], right)
        kB = rdma(send_k, recv_k, 1, comm_k.at[0], comm_k.at[3], left)
        vA = rdma(send_v, recv_v, 0, comm_v.at[0], comm_v.at[1], right)
        vB = rdma(send_v, recv_v, 1, comm_v.at[0], comm_v.at[3], left)
        kA.start(); vA.start(); kB.start(); vB.start()

        row_blk = lax.broadcasted_iota(jnp.int32, (S, S), 0) // BLK
        col_blk = lax.broadcasted_iota(jnp.int32, (S, S), 1) // BLK
        diag_mask = (col_blk <= row_blk).astype(jnp.float32)

        for b in range(B):
            q_buf[b] = jnp.dot(x_ref[b], wq_ref[...],
                               preferred_element_type=jnp.float32
                               ).astype(jnp.bfloat16)

        def attend(slot, b, keep=None, masked=False, init=False):
            for hh in range(HQ):
                lo = hh * DH
                q = q_buf[b, :, lo:lo + DH]
                kk = comm_k[slot, b, :, lo:lo + DH]
                vv = comm_v[slot, b, :, lo:lo + DH]
                sc = lax.dot_general(
                    q, kk, (((1,), (1,)), ((), ())),
                    preferred_element_type=jnp.float32) * SCALE
                w = jnp.exp(sc)
                if masked:
                    w = w * diag_mask
                if keep is not None:
                    w = w * keep
                pv = jnp.dot(w.astype(jnp.bfloat16), vv,
                             preferred_element_type=jnp.float32)
                ds = jnp.sum(w, axis=1, keepdims=True)
                if init:
                    acc[b, :, lo:lo + DH] = pv
                    den[b, hh] = ds
                else:
                    acc[b, :, lo:lo + DH] = acc[b, :, lo:lo + DH] + pv
                    den[b, hh] = den[b, hh] + ds

        def finish(b):
            for hh in range(HQ):
                lo = hh * DH
                ctx_buf[b, :, lo:lo + DH] = acc[b, :, lo:lo + DH] / den[b, hh]
            out_ref[b] = jnp.dot(ctx_buf[b], wo_ref[...],
                                 preferred_element_type=jnp.float32)

        for b in range(B):
            attend(0, b, masked=True, init=True)

        kA.wait_recv(); vA.wait_recv(); kB.wait_recv(); vB.wait_recv()
        kC = rdma(send_k, recv_k, 2, comm_k.at[1, 0], comm_k.at[2, 0], right)
        kD = rdma(send_k, recv_k, 3, comm_k.at[3, 1], comm_k.at[2, 1], left)
        vC = rdma(send_v, recv_v, 2, comm_v.at[1, 0], comm_v.at[2, 0], right)
        vD = rdma(send_v, recv_v, 3, comm_v.at[3, 1], comm_v.at[2, 1], left)
        kC.start(); vC.start(); kD.start(); vD.start()

        keep1 = (my >= 1).astype(jnp.float32)
        keep3 = (my == 3).astype(jnp.float32)
        keep2 = (my >= 2).astype(jnp.float32)
        for b in range(B):
            attend(1, b, keep=keep1)
            attend(3, b, keep=keep3)

        kC.wait_recv(); vC.wait_recv()
        attend(2, 0, keep=keep2)
        finish(0)
        kD.wait_recv(); vD.wait_recv()
        attend(2, 1, keep=keep2)
        finish(1)

        kA.wait_send(); vA.wait_send(); kB.wait_send(); vB.wait_send()
        kC.wait_send(); vC.wait_send(); kD.wait_send(); vD.wait_send()

    return pl.pallas_call(
        body,
        out_shape=jax.ShapeDtypeStruct((B, S, 768), jnp.float32),
        in_specs=[pl.BlockSpec(memory_space=pltpu.VMEM)] * 5,
        out_specs=pl.BlockSpec(memory_space=pltpu.VMEM),
        scratch_shapes=[
            pltpu.VMEM((N_DEV, B, S, HD), jnp.bfloat16),
            pltpu.VMEM((N_DEV, B, S, HD), jnp.bfloat16),
            pltpu.VMEM((B, S, HD), jnp.bfloat16),
            pltpu.VMEM((B, S, HD), jnp.float32),
            pltpu.VMEM((B, S, HD), jnp.float32),
            pltpu.VMEM((B, HQ, S, 1), jnp.float32),
            pltpu.SemaphoreType.DMA((4,)),
            pltpu.SemaphoreType.DMA((4,)),
            pltpu.SemaphoreType.DMA((4,)),
            pltpu.SemaphoreType.DMA((4,)),
        ],
        compiler_params=pltpu.CompilerParams(
            collective_id=0,
            vmem_limit_bytes=100 * 1024 * 1024,
        ),
    )(x, Wq, K2, V2, Wo)


# device time: 67841 ns/iter; 2.6524x vs baseline; 1.0208x over previous
import jax
import jax.numpy as jnp
from jax import lax
from jax.experimental import pallas as pl
from jax.experimental.pallas import tpu as pltpu

N_DEV = 4
B = 2
S = 512
HQ = 8
DH = 64
HD = HQ * DH
BLK = 64
SCALE = 0.125


def kernel(x, Wq, K_ext, V_ext, Wo):
    K2 = K_ext.reshape(B, S, HD)
    V2 = V_ext.reshape(B, S, HD)

    def body(x_ref, wq_ref, k_ref, v_ref, wo_ref, out_ref,
             comm_k, comm_v, q_buf, ctx_buf, acc, den,
             send_k, recv_k, send_v, recv_v):
        my = lax.axis_index("i")
        left = (my + N_DEV - 1) % N_DEV
        right = (my + 1) % N_DEV

        barrier = pltpu.get_barrier_semaphore()
        for nbr in (left, right):
            pl.semaphore_signal(barrier, inc=1, device_id=(nbr,),
                                device_id_type=pl.DeviceIdType.MESH)
        pl.semaphore_wait(barrier, 2)

        comm_k[0] = k_ref[...].astype(jnp.bfloat16)
        comm_v[0] = v_ref[...].astype(jnp.bfloat16)

        def rdma(sems_s, sems_r, i, src, dst, target):
            return pltpu.make_async_remote_copy(
                src_ref=src, dst_ref=dst,
                send_sem=sems_s.at[i], recv_sem=sems_r.at[i],
                device_id=(target,), device_id_type=pl.DeviceIdType.MESH)

        kA0 = rdma(send_k, recv_k, 0, comm_k.at[0, 0], comm_k.at[1, 0], right)
        kA1 = rdma(send_k, recv_k, 1, comm_k.at[0, 1], comm_k.at[1, 1], right)
        kB0 = rdma(send_k, recv_k, 2, comm_k.at[0, 0], comm_k.at[3, 0], left)
        kB1 = rdma(send_k, recv_k, 3, comm_k.at[0, 1], comm_k.at[3, 1], left)
        vA0 = rdma(send_v, recv_v, 0, comm_v.at[0, 0], comm_v.at[1, 0], right)
        vA1 = rdma(send_v, recv_v, 1, comm_v.at[0, 1], comm_v.at[1, 1], right)
        vB0 = rdma(send_v, recv_v, 2, comm_v.at[0, 0], comm_v.at[3, 0], left)
        vB1 = rdma(send_v, recv_v, 3, comm_v.at[0, 1], comm_v.at[3, 1], left)
        kA0.start(); vA0.start(); kB1.start(); vB1.start()
        kA1.start(); vA1.start(); kB0.start(); vB0.start()

        row_blk = lax.broadcasted_iota(jnp.int32, (S, S), 0) // BLK
        col_blk = lax.broadcasted_iota(jnp.int32, (S, S), 1) // BLK
        diag_mask = (col_blk <= row_blk).astype(jnp.float32)

        for b in range(B):
            q_buf[b] = jnp.dot(x_ref[b].astype(jnp.bfloat16),
                               wq_ref[...].astype(jnp.bfloat16),
                               preferred_element_type=jnp.float32
                               ).astype(jnp.bfloat16)

        def attend(slot, b, keep=None, masked=False, init=False):
            for hh in range(HQ):
                lo = hh * DH
                q = q_buf[b, :, lo:lo + DH]
                kk = comm_k[slot, b, :, lo:lo + DH]
                vv = comm_v[slot, b, :, lo:lo + DH]
                sc = lax.dot_general(
                    q, kk, (((1,), (1,)), ((), ())),
                    preferred_element_type=jnp.float32) * SCALE
                w = jnp.exp(sc)
                if masked:
                    w = w * diag_mask
                if keep is not None:
                    w = w * keep
                pv = jnp.dot(w.astype(jnp.bfloat16), vv,
                             preferred_element_type=jnp.float32)
                ds = jnp.sum(w, axis=1, keepdims=True)
                if init:
                    acc[b, :, lo:lo + DH] = pv
                    den[b, hh] = ds
                else:
                    acc[b, :, lo:lo + DH] = acc[b, :, lo:lo + DH] + pv
                    den[b, hh] = den[b, hh] + ds

        def finish(b):
            for hh in range(HQ):
                lo = hh * DH
                ctx_buf[b, :, lo:lo + DH] = (
                    acc[b, :, lo:lo + DH] / den[b, hh]).astype(jnp.bfloat16)
            out_ref[b] = jnp.dot(ctx_buf[b], wo_ref[...].astype(jnp.bfloat16),
                                 preferred_element_type=jnp.float32)

        for b in range(B):
            attend(0, b, masked=True, init=True)

        kA0.wait_recv(); vA0.wait_recv()
        kC = rdma(send_k, recv_k, 4, comm_k.at[1, 0], comm_k.at[2, 0], right)
        vC = rdma(send_v, recv_v, 4, comm_v.at[1, 0], comm_v.at[2, 0], right)
        kC.start(); vC.start()
        kB1.wait_recv(); vB1.wait_recv()
        kD = rdma(send_k, recv_k, 5, comm_k.at[3, 1], comm_k.at[2, 1], left)
        vD = rdma(send_v, recv_v, 5, comm_v.at[3, 1], comm_v.at[2, 1], left)
        kD.start(); vD.start()

        keep1 = (my >= 1).astype(jnp.float32)
        keep3 = (my == 3).astype(jnp.float32)
        keep2 = (my >= 2).astype(jnp.float32)
        attend(1, 0, keep=keep1)
        attend(3, 1, keep=keep3)
        kA1.wait_recv(); vA1.wait_recv()
        attend(1, 1, keep=keep1)
        kB0.wait_recv(); vB0.wait_recv()
        attend(3, 0, keep=keep3)

        kC.wait_recv(); vC.wait_recv()
        attend(2, 0, keep=keep2)
        finish(0)
        kD.wait_recv(); vD.wait_recv()
        attend(2, 1, keep=keep2)
        finish(1)

        kA0.wait_send(); vA0.wait_send(); kA1.wait_send(); vA1.wait_send()
        kB0.wait_send(); vB0.wait_send(); kB1.wait_send(); vB1.wait_send()
        kC.wait_send(); vC.wait_send(); kD.wait_send(); vD.wait_send()

    return pl.pallas_call(
        body,
        out_shape=jax.ShapeDtypeStruct((B, S, 768), jnp.float32),
        in_specs=[pl.BlockSpec(memory_space=pltpu.VMEM)] * 5,
        out_specs=pl.BlockSpec(memory_space=pltpu.VMEM),
        scratch_shapes=[
            pltpu.VMEM((N_DEV, B, S, HD), jnp.bfloat16),
            pltpu.VMEM((N_DEV, B, S, HD), jnp.bfloat16),
            pltpu.VMEM((B, S, HD), jnp.bfloat16),
            pltpu.VMEM((B, S, HD), jnp.bfloat16),
            pltpu.VMEM((B, S, HD), jnp.float32),
            pltpu.VMEM((B, HQ, S, 1), jnp.float32),
            pltpu.SemaphoreType.DMA((6,)),
            pltpu.SemaphoreType.DMA((6,)),
            pltpu.SemaphoreType.DMA((6,)),
            pltpu.SemaphoreType.DMA((6,)),
        ],
        compiler_params=pltpu.CompilerParams(
            collective_id=0,
            vmem_limit_bytes=100 * 1024 * 1024,
        ),
    )(x, Wq, K2, V2, Wo)
